# baseline (speedup 1.0000x reference)
import functools

import jax
import jax.numpy as jnp
from jax import lax
from jax.experimental import pallas as pl
from jax.experimental.pallas import tpu as pltpu
from jax.experimental.pallas import tpu_sc as plsc

EPS = 1e-12

_NUM_CORES = 2
_NUM_SUBCORES = 16
_NUM_TILES = _NUM_CORES * _NUM_SUBCORES


def _sc_gather(table, idx, n_rows, d):
    rows_per_tile = n_rows // _NUM_TILES
    chunk = 64
    nchunks = rows_per_tile // chunk

    mesh = plsc.VectorSubcoreMesh(core_axis_name="c", subcore_axis_name="s")

    @functools.partial(
        pl.kernel,
        mesh=mesh,
        out_type=jax.ShapeDtypeStruct((n_rows, d), table.dtype),
        scratch_types=[
            pltpu.VMEM((rows_per_tile,), jnp.int32),
            pltpu.VMEM((chunk, d), table.dtype),
            pltpu.VMEM((chunk, d), table.dtype),
            pltpu.SemaphoreType.DMA,
            pltpu.SemaphoreType.DMA,
        ],
    )
    def gather_kernel(table_hbm, idx_hbm, out_hbm, idx_v, rows0, rows1,
                      sem0, sem1):
        wid = lax.axis_index("s") * _NUM_CORES + lax.axis_index("c")
        base = wid * rows_per_tile
        pltpu.sync_copy(idx_hbm.at[pl.ds(base, rows_per_tile)], idx_v)
        bufs = (rows0, rows1)
        sems = (sem0, sem1)

        def start(g, b):
            pltpu.async_copy(
                table_hbm.at[idx_v.at[pl.ds(g * chunk, chunk)]],
                bufs[b], sems[b])

        def finish(g, b):
            pltpu.make_async_copy(
                table_hbm.at[idx_v.at[pl.ds(g * chunk, chunk)]],
                bufs[b], sems[b]).wait()
            pltpu.sync_copy(bufs[b],
                            out_hbm.at[pl.ds(base + g * chunk, chunk)])

        start(0, 0)

        @pl.loop(0, nchunks, step=2)
        def _(k):
            @pl.when(k + 1 < nchunks)
            def _():
                start(k + 1, 1)

            finish(k, 0)

            @pl.when(k + 2 < nchunks)
            def _():
                start(k + 2, 0)

            @pl.when(k + 1 < nchunks)
            def _():
                finish(k + 1, 1)

    return gather_kernel(table, idx)


def _ln_body(words_ref, pos_ref, w_ref, b_ref, o_ref):
    wi = words_ref[...]
    d2 = wi.shape[-1]
    d = 2 * d2
    lo = lax.bitcast_convert_type(wi << 16, jnp.float32)
    hi = lax.bitcast_convert_type(wi & jnp.int32(-65536), jnp.float32)
    pos = pos_ref[...][None]
    xlo = lo + pos[..., :d2]
    xhi = hi + pos[..., d2:]
    s = jnp.sum(xlo, -1, keepdims=True) + jnp.sum(xhi, -1, keepdims=True)
    mean = s * (1.0 / d)
    clo = xlo - mean
    chi = xhi - mean
    var = (jnp.sum(clo * clo, -1, keepdims=True)
           + jnp.sum(chi * chi, -1, keepdims=True)) * (1.0 / d)
    inv = lax.rsqrt(var + EPS)
    wv = w_ref[...][None]
    bv = b_ref[...][None]
    o_ref[..., :d2] = (clo * inv) * wv[..., :d2] + bv[..., :d2]
    o_ref[..., d2:] = (chi * inv) * wv[..., d2:] + bv[..., d2:]


def _ln_body_aliased(dst_ref, words_ref, pos_ref, w_ref, b_ref, o_ref):
    del dst_ref
    _ln_body(words_ref, pos_ref, w_ref, b_ref, o_ref)


def _pack_body(t_ref, o_ref):
    x = lax.bitcast_convert_type(t_ref[...], jnp.int32)
    d2 = x.shape[-1] // 2
    u = x[:, :d2]
    v = x[:, d2:]
    ru = u + 0x7FFF + (lax.shift_right_logical(u, 16) & 1)
    rv = v + 0x7FFF + (lax.shift_right_logical(v, 16) & 1)
    o_ref[...] = lax.shift_right_logical(ru, 16) | (rv & jnp.int32(-65536))


def _tc_pack(table):
    vocab, d = table.shape
    rows = 152
    grid = (vocab + rows - 1) // rows
    return pl.pallas_call(
        _pack_body,
        grid=(grid,),
        in_specs=[pl.BlockSpec((rows, d), lambda i: (i, 0))],
        out_specs=pl.BlockSpec((rows, d // 2), lambda i: (i, 0)),
        out_shape=jax.ShapeDtypeStruct((vocab, d // 2), jnp.int32),
    )(table)


_SB = 4


def _tc_add_ln_chunk(words_c, pos, w, b, out_buf, c, total_b):
    bsz_c, seq, dw = words_c.shape
    d = pos.shape[-1]
    nblk = bsz_c // _SB
    base = c * nblk

    word_spec = pl.BlockSpec((_SB, seq, dw), lambda i: (i, 0, 0))
    const_specs = [
        pl.BlockSpec((seq, d), lambda i: (0, 0)),
        pl.BlockSpec((1, d), lambda i: (0, 0)),
        pl.BlockSpec((1, d), lambda i: (0, 0)),
    ]
    out_spec = pl.BlockSpec((_SB, seq, d), lambda i, base=base: (base + i, 0, 0))
    out_shape = jax.ShapeDtypeStruct((total_b, seq, d), jnp.float32)

    if out_buf is None:
        return pl.pallas_call(
            _ln_body,
            grid=(nblk,),
            in_specs=[word_spec] + const_specs,
            out_specs=out_spec,
            out_shape=out_shape,
        )(words_c, pos, w, b)
    return pl.pallas_call(
        _ln_body_aliased,
        grid=(nblk,),
        in_specs=[pl.BlockSpec(memory_space=pl.ANY), word_spec]
        + const_specs,
        out_specs=out_spec,
        out_shape=out_shape,
        input_output_aliases={0: 0},
    )(out_buf, words_c, pos, w, b)


_NCHUNKS = 4


def kernel(input_ids, past_length, word_embeddings, position_embeddings,
           ln_weight, ln_bias):
    bsz, seq = input_ids.shape
    vocab, d = word_embeddings.shape
    n = bsz * seq

    max_pos = position_embeddings.shape[0]
    pos_ids = jnp.clip(jnp.arange(seq, dtype=jnp.int32) + past_length, 0,
                       max_pos - 1)
    pos = jnp.take(position_embeddings, pos_ids, axis=0)
    w2 = ln_weight.reshape(1, d)
    b2 = ln_bias.reshape(1, d)

    bsz_c = bsz // _NCHUNKS
    n_c = n // _NCHUNKS
    idx = input_ids.reshape(_NCHUNKS, n_c).astype(jnp.int32)

    table_packed = _tc_pack(word_embeddings)
    words = [
        _sc_gather(table_packed, idx[c], n_c, d // 2)
        .reshape(bsz_c, seq, d // 2)
        for c in range(_NCHUNKS)
    ]
    out = None
    for c in range(_NCHUNKS):
        out = _tc_add_ln_chunk(words[c], pos, w2, b2, out, c, bsz)
    return out


# speedup vs baseline: 1.2634x; 1.2634x over previous
import functools

import jax
import jax.numpy as jnp
from jax import lax
from jax.experimental import pallas as pl
from jax.experimental.pallas import tpu as pltpu
from jax.experimental.pallas import tpu_sc as plsc

EPS = 1e-12

_NUM_CORES = 2
_NUM_SUBCORES = 16
_NUM_TILES = _NUM_CORES * _NUM_SUBCORES


def _sc_gather(table, idx, n_rows, d):
    rows_per_tile = n_rows // _NUM_TILES
    chunk = 64
    nchunks = rows_per_tile // chunk

    mesh = plsc.VectorSubcoreMesh(core_axis_name="c", subcore_axis_name="s")

    @functools.partial(
        pl.kernel,
        mesh=mesh,
        out_type=jax.ShapeDtypeStruct((n_rows, d), table.dtype),
        scratch_types=[
            pltpu.VMEM((rows_per_tile,), jnp.int32),
            pltpu.VMEM((chunk, d), table.dtype),
            pltpu.VMEM((chunk, d), table.dtype),
            pltpu.SemaphoreType.DMA,
            pltpu.SemaphoreType.DMA,
        ],
    )
    def gather_kernel(table_hbm, idx_hbm, out_hbm, idx_v, rows0, rows1,
                      sem0, sem1):
        wid = lax.axis_index("s") * _NUM_CORES + lax.axis_index("c")
        base = wid * rows_per_tile
        pltpu.sync_copy(idx_hbm.at[pl.ds(base, rows_per_tile)], idx_v)
        bufs = (rows0, rows1)
        sems = (sem0, sem1)

        def start(g, b):
            pltpu.async_copy(
                table_hbm.at[idx_v.at[pl.ds(g * chunk, chunk)]],
                bufs[b], sems[b])

        def finish(g, b):
            pltpu.make_async_copy(
                table_hbm.at[idx_v.at[pl.ds(g * chunk, chunk)]],
                bufs[b], sems[b]).wait()
            pltpu.sync_copy(bufs[b],
                            out_hbm.at[pl.ds(base + g * chunk, chunk)])

        start(0, 0)

        @pl.loop(0, nchunks, step=2)
        def _(k):
            @pl.when(k + 1 < nchunks)
            def _():
                start(k + 1, 1)

            finish(k, 0)

            @pl.when(k + 2 < nchunks)
            def _():
                start(k + 2, 0)

            @pl.when(k + 1 < nchunks)
            def _():
                finish(k + 1, 1)

    return gather_kernel(table, idx)


def _ln_body(words_ref, pos_ref, w_ref, b_ref, o_ref):
    wi = words_ref[...]
    d2 = wi.shape[-1]
    d = 2 * d2
    lo = lax.bitcast_convert_type(wi << 16, jnp.float32)
    hi = lax.bitcast_convert_type(wi & jnp.int32(-65536), jnp.float32)
    pos = pos_ref[...][None]
    xlo = lo + pos[..., :d2]
    xhi = hi + pos[..., d2:]
    s = jnp.sum(xlo, -1, keepdims=True) + jnp.sum(xhi, -1, keepdims=True)
    mean = s * (1.0 / d)
    clo = xlo - mean
    chi = xhi - mean
    var = (jnp.sum(clo * clo, -1, keepdims=True)
           + jnp.sum(chi * chi, -1, keepdims=True)) * (1.0 / d)
    inv = lax.rsqrt(var + EPS)
    wv = w_ref[...][None]
    bv = b_ref[...][None]
    o_ref[..., :d2] = (clo * inv) * wv[..., :d2] + bv[..., :d2]
    o_ref[..., d2:] = (chi * inv) * wv[..., d2:] + bv[..., d2:]


def _ln_body_aliased(dst_ref, words_ref, pos_ref, w_ref, b_ref, o_ref):
    del dst_ref
    _ln_body(words_ref, pos_ref, w_ref, b_ref, o_ref)


def _pack_body(t_ref, o_ref):
    x = lax.bitcast_convert_type(t_ref[...], jnp.int32)
    d2 = x.shape[-1] // 2
    u = x[:, :d2]
    v = x[:, d2:]
    ru = u + 0x7FFF + (lax.shift_right_logical(u, 16) & 1)
    rv = v + 0x7FFF + (lax.shift_right_logical(v, 16) & 1)
    o_ref[...] = lax.shift_right_logical(ru, 16) | (rv & jnp.int32(-65536))


def _tc_pack(table):
    vocab, d = table.shape
    rows = 152
    grid = (vocab + rows - 1) // rows
    return pl.pallas_call(
        _pack_body,
        grid=(grid,),
        in_specs=[pl.BlockSpec((rows, d), lambda i: (i, 0))],
        out_specs=pl.BlockSpec((rows, d // 2), lambda i: (i, 0)),
        out_shape=jax.ShapeDtypeStruct((vocab, d // 2), jnp.int32),
    )(table)


_SB = 4


def _tc_add_ln_chunk(words_c, pos, w, b, out_buf, c, total_b):
    bsz_c, seq, dw = words_c.shape
    d = pos.shape[-1]
    nblk = bsz_c // _SB
    base = c * nblk

    word_spec = pl.BlockSpec((_SB, seq, dw), lambda i: (i, 0, 0))
    const_specs = [
        pl.BlockSpec((seq, d), lambda i: (0, 0)),
        pl.BlockSpec((1, d), lambda i: (0, 0)),
        pl.BlockSpec((1, d), lambda i: (0, 0)),
    ]
    out_spec = pl.BlockSpec((_SB, seq, d), lambda i, base=base: (base + i, 0, 0))
    out_shape = jax.ShapeDtypeStruct((total_b, seq, d), jnp.float32)

    if out_buf is None:
        return pl.pallas_call(
            _ln_body,
            grid=(nblk,),
            in_specs=[word_spec] + const_specs,
            out_specs=out_spec,
            out_shape=out_shape,
        )(words_c, pos, w, b)
    return pl.pallas_call(
        _ln_body_aliased,
        grid=(nblk,),
        in_specs=[pl.BlockSpec(memory_space=pl.ANY), word_spec]
        + const_specs,
        out_specs=out_spec,
        out_shape=out_shape,
        input_output_aliases={0: 0},
    )(out_buf, words_c, pos, w, b)


_NCHUNKS = 4


def kernel(input_ids, past_length, word_embeddings, position_embeddings,
           ln_weight, ln_bias):
    bsz, seq = input_ids.shape
    vocab, d = word_embeddings.shape
    n = bsz * seq

    max_pos = position_embeddings.shape[0]
    pos_ids = jnp.clip(jnp.arange(seq, dtype=jnp.int32) + past_length, 0,
                       max_pos - 1)
    pos = jnp.take(position_embeddings, pos_ids, axis=0)
    w2 = ln_weight.reshape(1, d)
    b2 = ln_bias.reshape(1, d)

    bsz_c = bsz // _NCHUNKS
    n_c = n // _NCHUNKS
    idx = input_ids.reshape(_NCHUNKS, n_c).astype(jnp.int32)

    u = lax.bitcast_convert_type(word_embeddings[:, :d // 2], jnp.int32)
    v = lax.bitcast_convert_type(word_embeddings[:, d // 2:], jnp.int32)
    ru = u + 0x7FFF + (lax.shift_right_logical(u, 16) & 1)
    rv = v + 0x7FFF + (lax.shift_right_logical(v, 16) & 1)
    table_packed = lax.shift_right_logical(ru, 16) | (rv & jnp.int32(-65536))
    words = [
        _sc_gather(table_packed, idx[c], n_c, d // 2)
        .reshape(bsz_c, seq, d // 2)
        for c in range(_NCHUNKS)
    ]
    out = None
    for c in range(_NCHUNKS):
        out = _tc_add_ln_chunk(words[c], pos, w2, b2, out, c, bsz)
    return out
